# baseline (speedup 1.0000x reference)
import math

import jax
import jax.numpy as jnp
from jax import lax
from jax.experimental import pallas as pl
from jax.experimental.pallas import tpu as pltpu

N_H = 12
N_I = 4
N_B = 4
D_MODEL = 768
D_K = 64
NB = 24
MAX_SEQ_LEN = 2048
MAXLEN = 2048
N_IS = N_I * (N_B + 1)
N_BS = N_B * N_I
N_TYPES_ITEM = N_B + 2
N_TYPES_INT = N_B + 1
NBLK = 512
KMAX = 11

_BF = jnp.bfloat16


def _dot(a, b):
    return jnp.dot(a.astype(_BF), b.astype(_BF),
                   preferred_element_type=jnp.float32)


def _get_cn_vec(x):
    inner = jnp.floor(
        jnp.log(jnp.maximum(4.0 * x / NB, 1e-20))
        / math.log(4.0 * MAX_SEQ_LEN / NB) * (NB / 4.0))
    f1 = NB / 4.0 + inner
    ub = NB / 2.0 - 1.0
    alt = (f1 + ub - jnp.abs(f1 - ub)) * 0.5
    return jnp.where(x < NB / 4.0, x, alt)


def _proj_body(item_ref, bseq_ref, wk_ref, wv_ref,
               kbs_ref, vbs_ref, kba_ref, vba_ref, wk_scr, wv_scr):
    @pl.when((pl.program_id(0) == 0) & (pl.program_id(1) == 0))
    def _cast_weights():
        wk_scr[...] = wk_ref[...].astype(_BF)
        wv_scr[...] = wv_ref[...].astype(_BF)

    x = item_ref[0].astype(_BF)
    bt = bseq_ref[0]
    kba = _dot(x, wk_scr[N_TYPES_ITEM - 1])
    vba = _dot(x, wv_scr[N_TYPES_ITEM - 1])
    acc_k = jnp.where(bt == float(N_TYPES_ITEM - 1), kba, 0.0)
    acc_v = jnp.where(bt == float(N_TYPES_ITEM - 1), vba, 0.0)
    zero = jnp.zeros_like(x)
    for t in range(N_TYPES_ITEM - 1):
        xm = jnp.where(bt == float(t), x, zero)
        acc_k = acc_k + _dot(xm, wk_scr[t])
        acc_v = acc_v + _dot(xm, wv_scr[t])
    kbs_ref[0] = acc_k.astype(_BF)
    vbs_ref[0] = acc_v.astype(_BF)
    kba_ref[0] = kba.astype(_BF)
    vba_ref[0] = vba.astype(_BF)


def _softmax_topk(s_raw, m, k_row):
    mrep = jnp.concatenate([m] * N_H, axis=1)
    s = jnp.where(mrep == 0, -1e30, s_raw * (1.0 / math.sqrt(D_K)))
    if k_row.shape[1] > 1:
        k_rep = jnp.concatenate([k_row] * N_H, axis=1)
    else:
        k_rep = k_row
    thr = jnp.full((1, s.shape[1]), jnp.inf, jnp.float32)
    cur = s
    mx = None
    for j in range(KMAX):
        mj = jnp.max(cur, axis=0, keepdims=True)
        if j == 0:
            mx = mj
        thr = jnp.where(k_rep == float(j + 1), mj, thr)
        cur = jnp.where(cur >= mj, -jnp.inf, cur)
    e = jnp.exp(s - mx)
    den = jnp.sum(e, axis=0, keepdims=True)
    return jnp.where(s >= thr, e, 0.0) / den


def _attn_body(intent_ref, bseq2_ref, wq_ref, kbs_ref, vbs_ref, kba_ref,
               vba_ref, maskt_ref, cntrep_ref, cntrow_ref,
               resbs_ref, resba_ref):
    xi = intent_ref[0]
    bt2 = bseq2_ref[0]
    zero = jnp.zeros_like(xi)
    xi5 = jnp.concatenate(
        [jnp.where(bt2 == float(t), xi, zero) for t in range(N_TYPES_INT)],
        axis=1)
    wq_flat = wq_ref[...].reshape(N_TYPES_INT * D_MODEL, N_H * D_K)
    q = _dot(xi5, wq_flat)
    qt = jnp.transpose(q).astype(_BF)

    rowh = lax.broadcasted_iota(jnp.int32, (D_MODEL, 1), 0) // D_K
    qt_bs = qt[:, 0:N_BS]
    qt_ba = qt[:, N_BS:N_IS]
    zb = jnp.zeros_like(qt_bs)
    za = jnp.zeros_like(qt_ba)
    qd_bs = jnp.concatenate(
        [jnp.where(rowh == h, qt_bs, zb) for h in range(N_H)], axis=1)
    qd_ba = jnp.concatenate(
        [jnp.where(rowh == h, qt_ba, za) for h in range(N_H)], axis=1)

    maskt = maskt_ref[0]
    k_bs = _get_cn_vec(cntrep_ref[0])
    k_ba = _get_cn_vec(jnp.sum(cntrow_ref[0], axis=1, keepdims=True))

    s_bs = _dot(kbs_ref[0], qd_bs)
    p_bs = _softmax_topk(s_bs, maskt[:, 0:N_BS], k_bs)
    resbs_ref[0] = _dot(jnp.transpose(p_bs), vbs_ref[0])

    s_ba = _dot(kba_ref[0], qd_ba)
    p_ba = _softmax_topk(s_ba, maskt[:, N_BS:N_IS], k_ba)
    resba_ref[0] = _dot(jnp.transpose(p_ba), vba_ref[0])


def kernel(item, intent, mask, b_seq, b_seq2, type_cnt, W_item, W_intent):
    bs = item.shape[0]
    hk = N_H * D_K
    wk = W_item[0].reshape(N_TYPES_ITEM, D_MODEL, hk)
    wv = W_item[1].reshape(N_TYPES_ITEM, D_MODEL, hk)
    wq = W_intent[0].reshape(N_TYPES_INT, D_MODEL, hk)
    bseq_f = b_seq.astype(jnp.float32)[..., None]
    bseq2_f = b_seq2.astype(jnp.float32)[..., None]
    mask_t = jnp.transpose(mask.reshape(bs, N_IS, MAXLEN), (0, 2, 1))
    cnt_rep = jnp.repeat(type_cnt.astype(jnp.float32), N_I, axis=1)[:, None, :]
    cnt_row = type_cnt.astype(jnp.float32)[:, None, :]

    nblks = MAXLEN // NBLK
    kv_shape = jax.ShapeDtypeStruct((bs, MAXLEN, hk), _BF)
    kbs, vbs, kba, vba = pl.pallas_call(
        _proj_body,
        grid=(bs, nblks),
        in_specs=[
            pl.BlockSpec((1, NBLK, D_MODEL), lambda b, n: (b, n, 0)),
            pl.BlockSpec((1, NBLK, 1), lambda b, n: (b, n, 0)),
            pl.BlockSpec((N_TYPES_ITEM, D_MODEL, hk), lambda b, n: (0, 0, 0)),
            pl.BlockSpec((N_TYPES_ITEM, D_MODEL, hk), lambda b, n: (0, 0, 0)),
        ],
        out_specs=[
            pl.BlockSpec((1, NBLK, hk), lambda b, n: (b, n, 0)),
            pl.BlockSpec((1, NBLK, hk), lambda b, n: (b, n, 0)),
            pl.BlockSpec((1, NBLK, hk), lambda b, n: (b, n, 0)),
            pl.BlockSpec((1, NBLK, hk), lambda b, n: (b, n, 0)),
        ],
        out_shape=[kv_shape, kv_shape, kv_shape, kv_shape],
        scratch_shapes=[pltpu.VMEM((N_TYPES_ITEM, D_MODEL, hk), _BF),
                        pltpu.VMEM((N_TYPES_ITEM, D_MODEL, hk), _BF)],
        compiler_params=pltpu.CompilerParams(vmem_limit_bytes=63 * 2**20),
    )(item, bseq_f, wk, wv)

    res_bs, res_ba = pl.pallas_call(
        _attn_body,
        grid=(bs,),
        in_specs=[
            pl.BlockSpec((1, N_IS, D_MODEL), lambda b: (0, 0, 0)),
            pl.BlockSpec((1, N_IS, 1), lambda b: (b, 0, 0)),
            pl.BlockSpec((N_TYPES_INT, D_MODEL, hk), lambda b: (0, 0, 0)),
            pl.BlockSpec((1, MAXLEN, hk), lambda b: (b, 0, 0)),
            pl.BlockSpec((1, MAXLEN, hk), lambda b: (b, 0, 0)),
            pl.BlockSpec((1, MAXLEN, hk), lambda b: (b, 0, 0)),
            pl.BlockSpec((1, MAXLEN, hk), lambda b: (b, 0, 0)),
            pl.BlockSpec((1, MAXLEN, N_IS), lambda b: (b, 0, 0)),
            pl.BlockSpec((1, 1, N_BS), lambda b: (b, 0, 0)),
            pl.BlockSpec((1, 1, N_B), lambda b: (b, 0, 0)),
        ],
        out_specs=[
            pl.BlockSpec((1, N_H * N_BS, hk), lambda b: (b, 0, 0)),
            pl.BlockSpec((1, N_H * N_I, hk), lambda b: (b, 0, 0)),
        ],
        out_shape=[
            jax.ShapeDtypeStruct((bs, N_H * N_BS, hk), jnp.float32),
            jax.ShapeDtypeStruct((bs, N_H * N_I, hk), jnp.float32),
        ],
        compiler_params=pltpu.CompilerParams(vmem_limit_bytes=63 * 2**20),
    )(intent, bseq2_f, wq, kbs, vbs, kba, vba, mask_t, cnt_rep, cnt_row)

    db = jnp.diagonal(res_bs.reshape(bs, N_H, N_BS, N_H, D_K),
                      axis1=1, axis2=3)
    da = jnp.diagonal(res_ba.reshape(bs, N_H, N_I, N_H, D_K),
                      axis1=1, axis2=3)
    x_bs = jnp.transpose(db, (0, 1, 3, 2)).reshape(bs, N_BS, hk)
    x_ba = jnp.transpose(da, (0, 1, 3, 2)).reshape(bs, N_I, hk)
    return jnp.concatenate([x_bs, x_ba], axis=1)


# speedup vs baseline: 1.1333x; 1.1333x over previous
import math

import jax
import jax.numpy as jnp
from jax import lax
from jax.experimental import pallas as pl
from jax.experimental.pallas import tpu as pltpu

N_H = 12
N_I = 4
N_B = 4
D_MODEL = 768
D_K = 64
NB = 24
MAX_SEQ_LEN = 2048
MAXLEN = 2048
N_IS = N_I * (N_B + 1)
N_BS = N_B * N_I
N_TYPES_ITEM = N_B + 2
N_TYPES_INT = N_B + 1
NBLK = 512
NBLKS = MAXLEN // NBLK
KMAX = 11

_BF = jnp.bfloat16


def _dot(a, b):
    return jnp.dot(a.astype(_BF), b.astype(_BF),
                   preferred_element_type=jnp.float32)


def _get_cn_vec(x):
    inner = jnp.floor(
        jnp.log(jnp.maximum(4.0 * x / NB, 1e-20))
        / math.log(4.0 * MAX_SEQ_LEN / NB) * (NB / 4.0))
    f1 = NB / 4.0 + inner
    ub = NB / 2.0 - 1.0
    alt = (f1 + ub - jnp.abs(f1 - ub)) * 0.5
    return jnp.where(x < NB / 4.0, x, alt)


def _softmax_topk(s_raw, m, k_row):
    mrep = jnp.concatenate([m] * N_H, axis=1)
    s = jnp.where(mrep == 0, -1e30, s_raw * (1.0 / math.sqrt(D_K)))
    if k_row.shape[1] > 1:
        k_rep = jnp.concatenate([k_row] * N_H, axis=1)
    else:
        k_rep = k_row
    thr = jnp.full((1, s.shape[1]), jnp.inf, jnp.float32)
    cur = s
    mx = None
    for j in range(KMAX):
        mj = jnp.max(cur, axis=0, keepdims=True)
        if j == 0:
            mx = mj
        thr = jnp.where(k_rep == float(j + 1), mj, thr)
        cur = jnp.where(cur >= mj, -jnp.inf, cur)
    e = jnp.exp(s - mx)
    den = jnp.sum(e, axis=0, keepdims=True)
    return jnp.where(s >= thr, e, 0.0) / den


def _body(item_ref, bseq_ref, wk_ref, wv_ref, intent_ref, bseq2_ref, wq_ref,
          maskt_ref, cntrep_ref, cntrow_ref, resbs_ref, resba_ref,
          kbs_s, vbs_s, kba_s, vba_s):
    n = pl.program_id(1)

    @pl.when(n < NBLKS)
    def _proj():
        x = item_ref[0].astype(_BF)
        bt = bseq_ref[0]
        kba = _dot(x, wk_ref[N_TYPES_ITEM - 1])
        vba = _dot(x, wv_ref[N_TYPES_ITEM - 1])
        acc_k = jnp.where(bt == float(N_TYPES_ITEM - 1), kba, 0.0)
        acc_v = jnp.where(bt == float(N_TYPES_ITEM - 1), vba, 0.0)
        zero = jnp.zeros_like(x)
        for t in range(N_TYPES_ITEM - 1):
            xm = jnp.where(bt == float(t), x, zero)
            acc_k = acc_k + _dot(xm, wk_ref[t])
            acc_v = acc_v + _dot(xm, wv_ref[t])
        rows = pl.ds(n * NBLK, NBLK)
        kbs_s[rows, :] = acc_k.astype(_BF)
        vbs_s[rows, :] = acc_v.astype(_BF)
        kba_s[rows, :] = kba.astype(_BF)
        vba_s[rows, :] = vba.astype(_BF)

    @pl.when(n == NBLKS)
    def _attn():
        xi = intent_ref[0]
        bt2 = bseq2_ref[0]
        zero = jnp.zeros_like(xi)
        xi5 = jnp.concatenate(
            [jnp.where(bt2 == float(t), xi, zero) for t in range(N_TYPES_INT)],
            axis=1)
        wq_flat = wq_ref[...].reshape(N_TYPES_INT * D_MODEL, N_H * D_K)
        q = _dot(xi5, wq_flat)
        qt = jnp.transpose(q).astype(_BF)

        rowh = lax.broadcasted_iota(jnp.int32, (D_MODEL, 1), 0) // D_K
        qt_bs = qt[:, 0:N_BS]
        qt_ba = qt[:, N_BS:N_IS]
        zb = jnp.zeros_like(qt_bs)
        za = jnp.zeros_like(qt_ba)
        qd_bs = jnp.concatenate(
            [jnp.where(rowh == h, qt_bs, zb) for h in range(N_H)], axis=1)
        qd_ba = jnp.concatenate(
            [jnp.where(rowh == h, qt_ba, za) for h in range(N_H)], axis=1)

        maskt = maskt_ref[0]
        k_bs = _get_cn_vec(cntrep_ref[0])
        k_ba = _get_cn_vec(jnp.sum(cntrow_ref[0], axis=1, keepdims=True))

        s_bs = _dot(kbs_s[...], qd_bs)
        p_bs = _softmax_topk(s_bs, maskt[:, 0:N_BS], k_bs)
        resbs_ref[0] = _dot(jnp.transpose(p_bs), vbs_s[...])

        s_ba = _dot(kba_s[...], qd_ba)
        p_ba = _softmax_topk(s_ba, maskt[:, N_BS:N_IS], k_ba)
        resba_ref[0] = _dot(jnp.transpose(p_ba), vba_s[...])


def kernel(item, intent, mask, b_seq, b_seq2, type_cnt, W_item, W_intent):
    bs = item.shape[0]
    hk = N_H * D_K
    wk = W_item[0].reshape(N_TYPES_ITEM, D_MODEL, hk).astype(_BF)
    wv = W_item[1].reshape(N_TYPES_ITEM, D_MODEL, hk).astype(_BF)
    wq = W_intent[0].reshape(N_TYPES_INT, D_MODEL, hk).astype(_BF)
    bseq_f = b_seq.astype(jnp.float32)[..., None]
    bseq2_f = b_seq2.astype(jnp.float32)[..., None]
    mask_t = jnp.transpose(mask.reshape(bs, N_IS, MAXLEN), (0, 2, 1))
    cnt_rep = jnp.repeat(type_cnt.astype(jnp.float32), N_I, axis=1)[:, None, :]
    cnt_row = type_cnt.astype(jnp.float32)[:, None, :]

    last = NBLKS - 1
    res_bs, res_ba = pl.pallas_call(
        _body,
        grid=(bs, NBLKS + 1),
        in_specs=[
            pl.BlockSpec((1, NBLK, D_MODEL),
                         lambda b, n: (b, jnp.minimum(n, last), 0)),
            pl.BlockSpec((1, NBLK, 1),
                         lambda b, n: (b, jnp.minimum(n, last), 0)),
            pl.BlockSpec((N_TYPES_ITEM, D_MODEL, hk), lambda b, n: (0, 0, 0)),
            pl.BlockSpec((N_TYPES_ITEM, D_MODEL, hk), lambda b, n: (0, 0, 0)),
            pl.BlockSpec((1, N_IS, D_MODEL), lambda b, n: (0, 0, 0)),
            pl.BlockSpec((1, N_IS, 1), lambda b, n: (b, 0, 0)),
            pl.BlockSpec((N_TYPES_INT, D_MODEL, hk), lambda b, n: (0, 0, 0)),
            pl.BlockSpec((1, MAXLEN, N_IS), lambda b, n: (b, 0, 0)),
            pl.BlockSpec((1, 1, N_BS), lambda b, n: (b, 0, 0)),
            pl.BlockSpec((1, 1, N_B), lambda b, n: (b, 0, 0)),
        ],
        out_specs=[
            pl.BlockSpec((1, N_H * N_BS, hk), lambda b, n: (b, 0, 0)),
            pl.BlockSpec((1, N_H * N_I, hk), lambda b, n: (b, 0, 0)),
        ],
        out_shape=[
            jax.ShapeDtypeStruct((bs, N_H * N_BS, hk), jnp.float32),
            jax.ShapeDtypeStruct((bs, N_H * N_I, hk), jnp.float32),
        ],
        scratch_shapes=[pltpu.VMEM((MAXLEN, hk), _BF) for _ in range(4)],
        compiler_params=pltpu.CompilerParams(vmem_limit_bytes=63 * 2**20),
    )(item, bseq_f, wk, wv, intent, bseq2_f, wq, mask_t, cnt_rep, cnt_row)

    db = jnp.diagonal(res_bs.reshape(bs, N_H, N_BS, N_H, D_K),
                      axis1=1, axis2=3)
    da = jnp.diagonal(res_ba.reshape(bs, N_H, N_I, N_H, D_K),
                      axis1=1, axis2=3)
    x_bs = jnp.transpose(db, (0, 1, 3, 2)).reshape(bs, N_BS, hk)
    x_ba = jnp.transpose(da, (0, 1, 3, 2)).reshape(bs, N_I, hk)
    return jnp.concatenate([x_bs, x_ba], axis=1)


# speedup vs baseline: 1.1375x; 1.0037x over previous
import math

import jax
import jax.numpy as jnp
from jax import lax
from jax.experimental import pallas as pl
from jax.experimental.pallas import tpu as pltpu

N_H = 12
N_I = 4
N_B = 4
D_MODEL = 768
D_K = 64
NB = 24
MAX_SEQ_LEN = 2048
MAXLEN = 2048
N_IS = N_I * (N_B + 1)
N_BS = N_B * N_I
N_TYPES_ITEM = N_B + 2
N_TYPES_INT = N_B + 1
NBLK = 512
NBLKS = MAXLEN // NBLK
KMAX = 11

_BF = jnp.bfloat16


def _dot(a, b):
    return jnp.dot(a.astype(_BF), b.astype(_BF),
                   preferred_element_type=jnp.float32)


def _get_cn_vec(x):
    inner = jnp.floor(
        jnp.log(jnp.maximum(4.0 * x / NB, 1e-20))
        / math.log(4.0 * MAX_SEQ_LEN / NB) * (NB / 4.0))
    f1 = NB / 4.0 + inner
    ub = NB / 2.0 - 1.0
    alt = (f1 + ub - jnp.abs(f1 - ub)) * 0.5
    return jnp.where(x < NB / 4.0, x, alt)


def _softmax_topk(s_raw, mrep, k_rep):
    s = jnp.where(mrep == 0, -1e30, s_raw * (1.0 / math.sqrt(D_K)))
    thr = jnp.full((1, s.shape[1]), jnp.inf, jnp.float32)
    cur = s
    mx = None
    for j in range(KMAX):
        mj = jnp.max(cur, axis=0, keepdims=True)
        if j == 0:
            mx = mj
        thr = jnp.where(k_rep == float(j + 1), mj, thr)
        cur = jnp.where(cur >= mj, -jnp.inf, cur)
    e = jnp.exp(s - mx)
    den = jnp.sum(e, axis=0, keepdims=True)
    return jnp.where(s >= thr, e, 0.0) / den


def _body(item_ref, bseq_ref, wk_ref, wv_ref, intent_ref, bseq2_ref, wq_ref,
          maskbs_ref, maskba_ref, cntrep_ref, cntrow_ref,
          resbs_ref, resba_ref, kbs_s, vbs_s, kba_s, vba_s):
    n = pl.program_id(1)

    @pl.when(n < NBLKS)
    def _proj():
        x = item_ref[0].astype(_BF)
        bt = bseq_ref[0]
        kba = _dot(x, wk_ref[N_TYPES_ITEM - 1])
        vba = _dot(x, wv_ref[N_TYPES_ITEM - 1])
        acc_k = jnp.where(bt == float(N_TYPES_ITEM - 1), kba, 0.0)
        acc_v = jnp.where(bt == float(N_TYPES_ITEM - 1), vba, 0.0)
        zero = jnp.zeros_like(x)
        for t in range(N_TYPES_ITEM - 1):
            xm = jnp.where(bt == float(t), x, zero)
            acc_k = acc_k + _dot(xm, wk_ref[t])
            acc_v = acc_v + _dot(xm, wv_ref[t])
        rows = pl.ds(n * NBLK, NBLK)
        kbs_s[rows, :] = acc_k.astype(_BF)
        vbs_s[rows, :] = acc_v.astype(_BF)
        kba_s[rows, :] = kba.astype(_BF)
        vba_s[rows, :] = vba.astype(_BF)

    @pl.when(n == NBLKS)
    def _attn():
        xi = intent_ref[0]
        bt2 = bseq2_ref[0]
        zero = jnp.zeros_like(xi)
        xi5 = jnp.concatenate(
            [jnp.where(bt2 == float(t), xi, zero) for t in range(N_TYPES_INT)],
            axis=1)
        wq_flat = wq_ref[...].reshape(N_TYPES_INT * D_MODEL, N_H * D_K)
        q = _dot(xi5, wq_flat)
        qt = jnp.transpose(q).astype(_BF)

        rowh = lax.broadcasted_iota(jnp.int32, (D_MODEL, 1), 0) // D_K
        qt_bs = qt[:, 0:N_BS]
        qt_ba = qt[:, N_BS:N_IS]
        zb = jnp.zeros_like(qt_bs)
        za = jnp.zeros_like(qt_ba)
        qd_bs = jnp.concatenate(
            [jnp.where(rowh == h, qt_bs, zb) for h in range(N_H)], axis=1)
        qd_ba = jnp.concatenate(
            [jnp.where(rowh == h, qt_ba, za) for h in range(N_H)], axis=1)

        k_bs = _get_cn_vec(cntrep_ref[0])
        k_ba = _get_cn_vec(jnp.sum(cntrow_ref[0], axis=1, keepdims=True))

        s_bs = _dot(kbs_s[...], qd_bs)
        p_bs = _softmax_topk(s_bs, maskbs_ref[0], k_bs)
        resbs_ref[0] = _dot(jnp.transpose(p_bs), vbs_s[...])

        s_ba = _dot(kba_s[...], qd_ba)
        p_ba = _softmax_topk(s_ba, maskba_ref[0], k_ba)
        resba_ref[0] = _dot(jnp.transpose(p_ba), vba_s[...])


def kernel(item, intent, mask, b_seq, b_seq2, type_cnt, W_item, W_intent):
    bs = item.shape[0]
    hk = N_H * D_K
    wk = W_item[0].reshape(N_TYPES_ITEM, D_MODEL, hk).astype(_BF)
    wv = W_item[1].reshape(N_TYPES_ITEM, D_MODEL, hk).astype(_BF)
    wq = W_intent[0].reshape(N_TYPES_INT, D_MODEL, hk).astype(_BF)
    bseq_f = b_seq.astype(jnp.float32)[..., None]
    bseq2_f = b_seq2.astype(jnp.float32)[..., None]
    mask_t = jnp.transpose(mask.reshape(bs, N_IS, MAXLEN), (0, 2, 1))
    mask_bs = jnp.tile(mask_t[:, :, 0:N_BS], (1, 1, N_H))
    mask_ba = jnp.tile(mask_t[:, :, N_BS:N_IS], (1, 1, N_H))
    cnt_rep = jnp.tile(
        jnp.repeat(type_cnt.astype(jnp.float32), N_I, axis=1)[:, None, :],
        (1, 1, N_H))
    cnt_row = type_cnt.astype(jnp.float32)[:, None, :]

    last = NBLKS - 1
    res_bs, res_ba = pl.pallas_call(
        _body,
        grid=(bs, NBLKS + 1),
        in_specs=[
            pl.BlockSpec((1, NBLK, D_MODEL),
                         lambda b, n: (b, jnp.minimum(n, last), 0)),
            pl.BlockSpec((1, NBLK, 1),
                         lambda b, n: (b, jnp.minimum(n, last), 0)),
            pl.BlockSpec((N_TYPES_ITEM, D_MODEL, hk), lambda b, n: (0, 0, 0)),
            pl.BlockSpec((N_TYPES_ITEM, D_MODEL, hk), lambda b, n: (0, 0, 0)),
            pl.BlockSpec((1, N_IS, D_MODEL), lambda b, n: (0, 0, 0)),
            pl.BlockSpec((1, N_IS, 1), lambda b, n: (b, 0, 0)),
            pl.BlockSpec((N_TYPES_INT, D_MODEL, hk), lambda b, n: (0, 0, 0)),
            pl.BlockSpec((1, MAXLEN, N_H * N_BS), lambda b, n: (b, 0, 0)),
            pl.BlockSpec((1, MAXLEN, N_H * N_I), lambda b, n: (b, 0, 0)),
            pl.BlockSpec((1, 1, N_H * N_BS), lambda b, n: (b, 0, 0)),
            pl.BlockSpec((1, 1, N_B), lambda b, n: (b, 0, 0)),
        ],
        out_specs=[
            pl.BlockSpec((1, N_H * N_BS, hk), lambda b, n: (b, 0, 0)),
            pl.BlockSpec((1, N_H * N_I, hk), lambda b, n: (b, 0, 0)),
        ],
        out_shape=[
            jax.ShapeDtypeStruct((bs, N_H * N_BS, hk), jnp.float32),
            jax.ShapeDtypeStruct((bs, N_H * N_I, hk), jnp.float32),
        ],
        scratch_shapes=[pltpu.VMEM((MAXLEN, hk), _BF) for _ in range(4)],
        compiler_params=pltpu.CompilerParams(vmem_limit_bytes=63 * 2**20),
    )(item, bseq_f, wk, wv, intent, bseq2_f, wq, mask_bs, mask_ba,
      cnt_rep, cnt_row)

    db = jnp.diagonal(res_bs.reshape(bs, N_H, N_BS, N_H, D_K),
                      axis1=1, axis2=3)
    da = jnp.diagonal(res_ba.reshape(bs, N_H, N_I, N_H, D_K),
                      axis1=1, axis2=3)
    x_bs = jnp.transpose(db, (0, 1, 3, 2)).reshape(bs, N_BS, hk)
    x_ba = jnp.transpose(da, (0, 1, 3, 2)).reshape(bs, N_I, hk)
    return jnp.concatenate([x_bs, x_ba], axis=1)


# speedup vs baseline: 1.1453x; 1.0068x over previous
import math

import jax
import jax.numpy as jnp
from jax import lax
from jax.experimental import pallas as pl
from jax.experimental.pallas import tpu as pltpu

N_H = 12
N_I = 4
N_B = 4
D_MODEL = 768
D_K = 64
NB = 24
MAX_SEQ_LEN = 2048
MAXLEN = 2048
N_IS = N_I * (N_B + 1)
N_BS = N_B * N_I
N_TYPES_ITEM = N_B + 2
N_TYPES_INT = N_B + 1
NBLK = 1024
NBLKS = MAXLEN // NBLK
KMAX = 11

_BF = jnp.bfloat16


def _dot(a, b):
    return jnp.dot(a.astype(_BF), b.astype(_BF),
                   preferred_element_type=jnp.float32)


def _get_cn_vec(x):
    inner = jnp.floor(
        jnp.log(jnp.maximum(4.0 * x / NB, 1e-20))
        / math.log(4.0 * MAX_SEQ_LEN / NB) * (NB / 4.0))
    f1 = NB / 4.0 + inner
    ub = NB / 2.0 - 1.0
    alt = (f1 + ub - jnp.abs(f1 - ub)) * 0.5
    return jnp.where(x < NB / 4.0, x, alt)


def _softmax_topk(s_raw, mrep, k_rep):
    s = jnp.where(mrep == 0, -1e30, s_raw * (1.0 / math.sqrt(D_K)))
    thr = jnp.full((1, s.shape[1]), jnp.inf, jnp.float32)
    cur = s
    mx = None
    for j in range(KMAX):
        mj = jnp.max(cur, axis=0, keepdims=True)
        if j == 0:
            mx = mj
        thr = jnp.where(k_rep == float(j + 1), mj, thr)
        cur = jnp.where(cur >= mj, -jnp.inf, cur)
    e = jnp.exp(s - mx)
    den = jnp.sum(e, axis=0, keepdims=True)
    return jnp.where(s >= thr, e, 0.0) / den


def _body(item_ref, bseq_ref, wk_ref, wv_ref, intent_ref, bseq2_ref, wq_ref,
          maskbs_ref, maskba_ref, cntrep_ref, cntrow_ref,
          resbs_ref, resba_ref, kbs_s, vbs_s, kba_s, vba_s):
    n = pl.program_id(1)

    @pl.when(n < NBLKS)
    def _proj():
        x = item_ref[0].astype(_BF)
        bt = bseq_ref[0]
        kba = _dot(x, wk_ref[N_TYPES_ITEM - 1])
        vba = _dot(x, wv_ref[N_TYPES_ITEM - 1])
        acc_k = jnp.where(bt == float(N_TYPES_ITEM - 1), kba, 0.0)
        acc_v = jnp.where(bt == float(N_TYPES_ITEM - 1), vba, 0.0)
        zero = jnp.zeros_like(x)
        for t in range(N_TYPES_ITEM - 1):
            xm = jnp.where(bt == float(t), x, zero)
            acc_k = acc_k + _dot(xm, wk_ref[t])
            acc_v = acc_v + _dot(xm, wv_ref[t])
        rows = pl.ds(n * NBLK, NBLK)
        kbs_s[rows, :] = acc_k.astype(_BF)
        vbs_s[rows, :] = acc_v.astype(_BF)
        kba_s[rows, :] = kba.astype(_BF)
        vba_s[rows, :] = vba.astype(_BF)

    @pl.when(n == NBLKS)
    def _attn():
        xi = intent_ref[0]
        bt2 = bseq2_ref[0]
        zero = jnp.zeros_like(xi)
        xi5 = jnp.concatenate(
            [jnp.where(bt2 == float(t), xi, zero) for t in range(N_TYPES_INT)],
            axis=1)
        wq_flat = wq_ref[...].reshape(N_TYPES_INT * D_MODEL, N_H * D_K)
        q = _dot(xi5, wq_flat)
        qt = jnp.transpose(q).astype(_BF)

        rowh = lax.broadcasted_iota(jnp.int32, (D_MODEL, 1), 0) // D_K
        qt_bs = qt[:, 0:N_BS]
        qt_ba = qt[:, N_BS:N_IS]
        zb = jnp.zeros_like(qt_bs)
        za = jnp.zeros_like(qt_ba)
        qd_bs = jnp.concatenate(
            [jnp.where(rowh == h, qt_bs, zb) for h in range(N_H)], axis=1)
        qd_ba = jnp.concatenate(
            [jnp.where(rowh == h, qt_ba, za) for h in range(N_H)], axis=1)

        k_bs = _get_cn_vec(cntrep_ref[0])
        k_ba = _get_cn_vec(jnp.sum(cntrow_ref[0], axis=1, keepdims=True))

        s_bs = _dot(kbs_s[...], qd_bs)
        p_bs = _softmax_topk(s_bs, maskbs_ref[0], k_bs)
        resbs_ref[0] = _dot(jnp.transpose(p_bs), vbs_s[...])

        s_ba = _dot(kba_s[...], qd_ba)
        p_ba = _softmax_topk(s_ba, maskba_ref[0], k_ba)
        resba_ref[0] = _dot(jnp.transpose(p_ba), vba_s[...])


def kernel(item, intent, mask, b_seq, b_seq2, type_cnt, W_item, W_intent):
    bs = item.shape[0]
    hk = N_H * D_K
    wk = W_item[0].reshape(N_TYPES_ITEM, D_MODEL, hk).astype(_BF)
    wv = W_item[1].reshape(N_TYPES_ITEM, D_MODEL, hk).astype(_BF)
    wq = W_intent[0].reshape(N_TYPES_INT, D_MODEL, hk).astype(_BF)
    bseq_f = b_seq.astype(jnp.float32)[..., None]
    bseq2_f = b_seq2.astype(jnp.float32)[..., None]
    mask_t = jnp.transpose(mask.reshape(bs, N_IS, MAXLEN), (0, 2, 1))
    mask_bs = jnp.tile(mask_t[:, :, 0:N_BS], (1, 1, N_H))
    mask_ba = jnp.tile(mask_t[:, :, N_BS:N_IS], (1, 1, N_H))
    cnt_rep = jnp.tile(
        jnp.repeat(type_cnt.astype(jnp.float32), N_I, axis=1)[:, None, :],
        (1, 1, N_H))
    cnt_row = type_cnt.astype(jnp.float32)[:, None, :]

    last = NBLKS - 1
    res_bs, res_ba = pl.pallas_call(
        _body,
        grid=(bs, NBLKS + 1),
        in_specs=[
            pl.BlockSpec((1, NBLK, D_MODEL),
                         lambda b, n: (b, jnp.minimum(n, last), 0)),
            pl.BlockSpec((1, NBLK, 1),
                         lambda b, n: (b, jnp.minimum(n, last), 0)),
            pl.BlockSpec((N_TYPES_ITEM, D_MODEL, hk), lambda b, n: (0, 0, 0)),
            pl.BlockSpec((N_TYPES_ITEM, D_MODEL, hk), lambda b, n: (0, 0, 0)),
            pl.BlockSpec((1, N_IS, D_MODEL), lambda b, n: (0, 0, 0)),
            pl.BlockSpec((1, N_IS, 1), lambda b, n: (b, 0, 0)),
            pl.BlockSpec((N_TYPES_INT, D_MODEL, hk), lambda b, n: (0, 0, 0)),
            pl.BlockSpec((1, MAXLEN, N_H * N_BS), lambda b, n: (b, 0, 0)),
            pl.BlockSpec((1, MAXLEN, N_H * N_I), lambda b, n: (b, 0, 0)),
            pl.BlockSpec((1, 1, N_H * N_BS), lambda b, n: (b, 0, 0)),
            pl.BlockSpec((1, 1, N_B), lambda b, n: (b, 0, 0)),
        ],
        out_specs=[
            pl.BlockSpec((1, N_H * N_BS, hk), lambda b, n: (b, 0, 0)),
            pl.BlockSpec((1, N_H * N_I, hk), lambda b, n: (b, 0, 0)),
        ],
        out_shape=[
            jax.ShapeDtypeStruct((bs, N_H * N_BS, hk), jnp.float32),
            jax.ShapeDtypeStruct((bs, N_H * N_I, hk), jnp.float32),
        ],
        scratch_shapes=[pltpu.VMEM((MAXLEN, hk), _BF) for _ in range(4)],
        compiler_params=pltpu.CompilerParams(vmem_limit_bytes=63 * 2**20),
    )(item, bseq_f, wk, wv, intent, bseq2_f, wq, mask_bs, mask_ba,
      cnt_rep, cnt_row)

    db = jnp.diagonal(res_bs.reshape(bs, N_H, N_BS, N_H, D_K),
                      axis1=1, axis2=3)
    da = jnp.diagonal(res_ba.reshape(bs, N_H, N_I, N_H, D_K),
                      axis1=1, axis2=3)
    x_bs = jnp.transpose(db, (0, 1, 3, 2)).reshape(bs, N_BS, hk)
    x_ba = jnp.transpose(da, (0, 1, 3, 2)).reshape(bs, N_I, hk)
    return jnp.concatenate([x_bs, x_ba], axis=1)
